# baseline (speedup 1.0000x reference)
import jax
import jax.numpy as jnp
from jax.experimental import pallas as pl
from jax.experimental.pallas import tpu as pltpu

_N, _C, _H, _W = 1024, 256, 28, 28
_P = _H * _W
_NS = 4


def _mean_body(zt_ref, out_ref):
    w = None
    for j in range(4):
        for i in range(4):
            t = zt_ref[i, j]
            w = t if w is None else w + t
    wi, wj = pl.program_id(1), pl.program_id(2)
    first = (wi == 0) & (wj == 0)
    last = (wi == 6) & (wj == 6)

    @pl.when(first)
    def _init():
        out_ref[...] = w

    @pl.when(~first & ~last)
    def _accum():
        out_ref[...] = out_ref[...] + w

    @pl.when(last)
    def _final():
        out_ref[...] = (out_ref[...] + w) * jnp.float32(1.0 / _P)


def _spatial_mean(z):
    zt = jnp.transpose(z, (2, 3, 0, 1))
    nb = _N // _NS
    return pl.pallas_call(
        _mean_body,
        grid=(_NS, 7, 7),
        in_specs=[pl.BlockSpec((4, 4, nb, _C), lambda n, a, b: (a, b, n, 0))],
        out_specs=pl.BlockSpec((nb, _C), lambda n, a, b: (n, 0)),
        out_shape=jax.ShapeDtypeStruct((_N, _C), jnp.float32),
        compiler_params=pltpu.CompilerParams(
            dimension_semantics=("arbitrary", "arbitrary", "arbitrary")),
    )(zt)


def _thermo(h):
    h = jnp.flip(h, -1)
    s = jnp.cumsum(h, -1)
    return jnp.flip(s, -1)


def _bn_train(x, gamma, beta, eps=1e-5):
    mu = x.mean(0)
    var = x.var(0)
    return gamma * (x - mu) / jnp.sqrt(var + eps) + beta


def kernel(z, SNR, W1, b1, g1, be1, W2, b2, g2, be2, W3, b3, temp):
    feat = jnp.concatenate([_spatial_mean(z), SNR], axis=-1)
    h = feat @ W1.T + b1
    h = jax.nn.relu(h)
    h = _bn_train(h, g1, be1)
    h = h @ W2.T + b2
    h = jax.nn.relu(h)
    h = _bn_train(h, g2, be2)
    logits = h @ W3.T + b3
    g = jax.random.gumbel(jax.random.key(42), logits.shape, dtype=logits.dtype)
    soft = jax.nn.softmax((logits + g) / temp, axis=-1)
    index = jax.nn.one_hot(jnp.argmax(soft, axis=-1), soft.shape[-1], dtype=soft.dtype)
    bias = jax.lax.stop_gradient(index - soft)
    hard = soft + bias
    soft_mask = _thermo(soft[:, 1:])
    hard_mask = _thermo(hard[:, 1:])
    return (hard_mask, soft_mask, logits)


# speedup vs baseline: 1.1111x; 1.1111x over previous
import jax
import jax.numpy as jnp
from jax.experimental import pallas as pl
from jax.experimental.pallas import tpu as pltpu

_N, _C, _H, _W = 1024, 256, 28, 28
_P = _H * _W
_NS = 2


def _mean_body(zt_ref, out_ref):
    w = None
    for j in range(4):
        for i in range(4):
            t = zt_ref[i, j]
            w = t if w is None else w + t
    wi, wj = pl.program_id(1), pl.program_id(2)
    first = (wi == 0) & (wj == 0)
    last = (wi == 6) & (wj == 6)

    @pl.when(first)
    def _init():
        out_ref[...] = w

    @pl.when(~first & ~last)
    def _accum():
        out_ref[...] = out_ref[...] + w

    @pl.when(last)
    def _final():
        out_ref[...] = (out_ref[...] + w) * jnp.float32(1.0 / _P)


def _spatial_mean(z):
    zt = jnp.transpose(z, (2, 3, 0, 1))
    nb = _N // _NS
    return pl.pallas_call(
        _mean_body,
        grid=(_NS, 7, 7),
        in_specs=[pl.BlockSpec((4, 4, nb, _C), lambda n, a, b: (a, b, n, 0))],
        out_specs=pl.BlockSpec((nb, _C), lambda n, a, b: (n, 0)),
        out_shape=jax.ShapeDtypeStruct((_N, _C), jnp.float32),
        compiler_params=pltpu.CompilerParams(
            dimension_semantics=("arbitrary", "arbitrary", "arbitrary")),
    )(zt)


def _thermo(h):
    h = jnp.flip(h, -1)
    s = jnp.cumsum(h, -1)
    return jnp.flip(s, -1)


def _bn_train(x, gamma, beta, eps=1e-5):
    mu = x.mean(0)
    var = x.var(0)
    return gamma * (x - mu) / jnp.sqrt(var + eps) + beta


def kernel(z, SNR, W1, b1, g1, be1, W2, b2, g2, be2, W3, b3, temp):
    feat = jnp.concatenate([_spatial_mean(z), SNR], axis=-1)
    h = feat @ W1.T + b1
    h = jax.nn.relu(h)
    h = _bn_train(h, g1, be1)
    h = h @ W2.T + b2
    h = jax.nn.relu(h)
    h = _bn_train(h, g2, be2)
    logits = h @ W3.T + b3
    g = jax.random.gumbel(jax.random.key(42), logits.shape, dtype=logits.dtype)
    soft = jax.nn.softmax((logits + g) / temp, axis=-1)
    index = jax.nn.one_hot(jnp.argmax(soft, axis=-1), soft.shape[-1], dtype=soft.dtype)
    bias = jax.lax.stop_gradient(index - soft)
    hard = soft + bias
    soft_mask = _thermo(soft[:, 1:])
    hard_mask = _thermo(hard[:, 1:])
    return (hard_mask, soft_mask, logits)
